# baseline (speedup 1.0000x reference)
import functools

import jax
import jax.numpy as jnp
from jax import lax
from jax.experimental import pallas as pl
from jax.experimental.pallas import tpu as pltpu
from jax.experimental.pallas import tpu_sc as plsc

N_NODES = 10000
NP = 10240
E_RAW = 320000
E_TOT = E_RAW + N_NODES
NWORK = 32
EDGE_BLK = 128
NBLK = 82
CHUNK = NBLK * EDGE_BLK
E_PAD = CHUNK * NWORK
NEG = -1e30
D_FEAT = 128
NHID = 64
N_CLASSES = 16


WF = 128


def _make_sc_edge_kernel(nscale):
    mesh = plsc.VectorSubcoreMesh(core_axis_name="c", subcore_axis_name="s")
    rows_per_sub = NP // 16

    @functools.partial(
        pl.kernel,
        out_type=jax.ShapeDtypeStruct((2, NP, WF), jnp.float32),
        mesh=mesh,
        scratch_types=[
            pltpu.VMEM((EDGE_BLK,), jnp.int32),
            pltpu.VMEM((EDGE_BLK,), jnp.int32),
            pltpu.VMEM((EDGE_BLK,), jnp.int32),
            pltpu.VMEM((EDGE_BLK,), jnp.int32),
            pltpu.VMEM((EDGE_BLK,), jnp.float32),
            pltpu.VMEM((EDGE_BLK,), jnp.float32),
            pltpu.VMEM((EDGE_BLK,), jnp.float32),
            pltpu.VMEM((EDGE_BLK,), jnp.float32),
            pltpu.VMEM((EDGE_BLK,), jnp.float32),
            pltpu.VMEM((EDGE_BLK, WF), jnp.float32),
            pltpu.VMEM((EDGE_BLK, WF), jnp.float32),
            pltpu.VMEM_SHARED((NP,), jnp.float32),
            pltpu.VMEM_SHARED((NP,), jnp.float32),
            pltpu.VMEM_SHARED((NP, WF), jnp.float32),
            pltpu.SemaphoreType.DMA,
            pltpu.SemaphoreType.DMA,
            pltpu.SemaphoreType.DMA,
            pltpu.SemaphoreType.DMA,
            pltpu.SemaphoreType.DMA,
            pltpu.SemaphoreType.DMA,
            pltpu.SemaphoreType.DMA,
            pltpu.SemaphoreType.DMA,
        ],
        compiler_params=pltpu.CompilerParams(needs_layout_passes=False),
    )
    def k(src_hbm, dst_hbm, s_hbm, d_hbm, hext_hbm, acc_hbm,
          sidx0, sidx1, didx0, didx1, sv0, sv1, dv0, dv1, w_v,
          rows0, rows1, s_sh, d_sh, acc_sh,
          isem0, isem1, csem0, csem1, gsem0, gsem1, ssem0, ssem1):
        cid = lax.axis_index("c")
        sid = lax.axis_index("s")
        wid = sid * 2 + cid
        sidx = (sidx0, sidx1)
        didx = (didx0, didx1)
        sv = (sv0, sv1)
        dv = (dv0, dv1)
        rows = (rows0, rows1)
        isem = (isem0, isem1)
        csem = (csem0, csem1)
        gsem = (gsem0, gsem1)
        ssem = (ssem0, ssem1)

        @pl.when(sid == 0)
        def _():
            pltpu.sync_copy(s_hbm, s_sh)
            pltpu.sync_copy(d_hbm, d_sh)

        @pl.loop(0, EDGE_BLK)
        def _(r):
            for c in range(WF // 16):
                rows0[r, pl.ds(c * 16, 16)] = jnp.zeros((16,), jnp.float32)

        @pl.loop(0, rows_per_sub, step=EDGE_BLK)
        def _(r0):
            pltpu.sync_copy(
                rows0, acc_sh.at[pl.ds(sid * rows_per_sub + r0, EDGE_BLK)])

        plsc.subcore_barrier()

        def issue_idx(tt, b):
            pltpu.async_copy(src_hbm.at[wid].at[tt], sidx[b], isem[b])
            pltpu.async_copy(dst_hbm.at[wid].at[tt], didx[b], isem[b])

        def wait_idx(tt, b):
            pltpu.make_async_copy(
                src_hbm.at[wid].at[tt], sidx[b], isem[b]).wait()
            pltpu.make_async_copy(
                dst_hbm.at[wid].at[tt], didx[b], isem[b]).wait()

        def issue_scores(b):
            pltpu.async_copy(s_sh.at[sidx[b]], sv[b], csem[b])
            pltpu.async_copy(d_sh.at[didx[b]], dv[b], csem[b])

        def wait_scores(b):
            pltpu.make_async_copy(s_sh.at[sidx[b]], sv[b], csem[b]).wait()
            pltpu.make_async_copy(d_sh.at[didx[b]], dv[b], csem[b]).wait()

        def issue_gather(b):
            pltpu.async_copy(hext_hbm.at[sidx[b]], rows[b], gsem[b])

        def wait_gather(b):
            pltpu.make_async_copy(
                hext_hbm.at[sidx[b]], rows[b], gsem[b]).wait()

        def issue_scatter(b):
            pltpu.async_copy(rows[b], acc_sh.at[didx[b]], ssem[b], add=True)

        def wait_scatter(b):
            pltpu.make_async_copy(
                rows[b], acc_sh.at[didx[b]], ssem[b]).wait()

        def block_body(tt, b, first=False, last=False):
            ob = 1 - b
            if not first:
                wait_scatter(ob)
            if not last:
                issue_idx(tt + 1, ob)
            wait_scores(b)

            @pl.loop(0, EDGE_BLK, step=16)
            def _(j):
                e = sv[b][pl.ds(j, 16)] + dv[b][pl.ds(j, 16)]
                e = jnp.where(e >= 0.0, e, 0.2 * e)
                w_v[pl.ds(j, 16)] = jnp.exp(e)

            wait_gather(b)
            if not last:
                wait_idx(tt + 1, ob)
                issue_scores(ob)
                issue_gather(ob)

            @pl.loop(0, EDGE_BLK, step=4)
            def _(r):
                for u in range(4):
                    wv = plsc.load_gather(
                        w_v, [jnp.full((16,), r + u, jnp.int32)])
                    for c in range(nscale):
                        sl = pl.ds(c * 16, 16)
                        rows[b][r + u, sl] = rows[b][r + u, sl] * wv

            issue_scatter(b)

        issue_idx(0, 0)
        wait_idx(0, 0)
        issue_scores(0)
        issue_gather(0)

        block_body(0, 0, first=True)

        @pl.loop(1, NBLK - 1, step=2)
        def _(t):
            block_body(t, 1)
            block_body(t + 1, 0)

        block_body(NBLK - 1, 1, last=True)
        wait_scatter(1)

        plsc.subcore_barrier()

        pltpu.sync_copy(
            acc_sh.at[pl.ds(sid * rows_per_sub, rows_per_sub)],
            acc_hbm.at[cid].at[pl.ds(sid * rows_per_sub, rows_per_sub)])

    return k


_sc_edge_wide = _make_sc_edge_kernel(5)
_sc_edge_narrow = _make_sc_edge_kernel(2)


def _valid_mask():
    return lax.broadcasted_iota(jnp.int32, (NP, 1), 0) < N_NODES


def _emit_layer_outputs(h, f_out, hext_ref, sd_ref, a):
    valid = _valid_mask()
    sd = jnp.dot(h, a, preferred_element_type=jnp.float32)
    hext_ref[:, :f_out] = jnp.where(valid, h, 0.0)
    col = lax.broadcasted_iota(jnp.int32, (NP, WF - f_out), 1)
    hext_ref[:, f_out:] = jnp.where(valid & (col == 0), 1.0, 0.0)
    sd_ref[...] = jnp.where(valid, sd, NEG)


def _dense1_body(xp_ref, w_ref, a_ref, hext_ref, sd_ref):
    h = jnp.dot(xp_ref[...], w_ref[...], preferred_element_type=jnp.float32)
    _emit_layer_outputs(h, NHID, hext_ref, sd_ref, a_ref[...])


def _make_combine_body(f_in, f_out):
    def body(acc_ref, b_ref, w_ref, a_ref, hext_ref, sd_ref):
        g = acc_ref[0] + acc_ref[1]
        num = g[:, :f_in]
        den = g[:, f_in:f_in + 1] + 1e-16
        xn = jnp.maximum(num / den + b_ref[...], 0.0)
        xn = jnp.where(_valid_mask(), xn, 0.0)
        h = jnp.dot(xn, w_ref[...], preferred_element_type=jnp.float32)
        _emit_layer_outputs(h, f_out, hext_ref, sd_ref, a_ref[...])
    return body


def _final_body(acc_ref, b_ref, out_ref):
    g = acc_ref[0] + acc_ref[1]
    o = g[:, :N_CLASSES] / (g[:, N_CLASSES:N_CLASSES + 1] + 1e-16) + b_ref[...]
    m = jnp.max(o, axis=1, keepdims=True)
    z = o - m
    out_ref[...] = z - jnp.log(jnp.sum(jnp.exp(z), axis=1, keepdims=True))


def _f32(shape):
    return jax.ShapeDtypeStruct(shape, jnp.float32)


def kernel(x, edge_index, W1, a1s, a1d, b1, W2, a2s, a2d, b2,
           W3, a3s, a3d, b3):
    ei = edge_index.astype(jnp.int32)
    loops = jnp.arange(N_NODES, dtype=jnp.int32)
    pad_src = jnp.full((E_PAD - E_TOT,), N_NODES, jnp.int32)
    pad_dst = N_NODES + (jnp.arange(E_PAD - E_TOT, dtype=jnp.int32)
                         % (NP - N_NODES))
    src = jnp.concatenate([ei[0], loops, pad_src]).reshape(
        NWORK, NBLK, EDGE_BLK)
    dst = jnp.concatenate([ei[1], loops, pad_dst]).reshape(
        NWORK, NBLK, EDGE_BLK)

    xp = jnp.pad(x, ((0, NP - N_NODES), (0, 0)))
    A1 = jnp.stack([a1s, a1d], axis=1)
    A2 = jnp.stack([a2s, a2d], axis=1)
    A3 = jnp.stack([a3s, a3d], axis=1)

    hext1, sd1 = pl.pallas_call(
        _dense1_body, out_shape=(_f32((NP, WF)), _f32((NP, 2))),
    )(xp, W1, A1)
    acc1 = _sc_edge_wide(src, dst, sd1[:, 0], sd1[:, 1], hext1)

    hext2, sd2 = pl.pallas_call(
        _make_combine_body(NHID, NHID),
        out_shape=(_f32((NP, WF)), _f32((NP, 2))),
    )(acc1, b1.reshape(1, -1), W2, A2)
    acc2 = _sc_edge_wide(src, dst, sd2[:, 0], sd2[:, 1], hext2)

    hext3, sd3 = pl.pallas_call(
        _make_combine_body(NHID, N_CLASSES),
        out_shape=(_f32((NP, WF)), _f32((NP, 2))),
    )(acc2, b2.reshape(1, -1), W3, A3)
    acc3 = _sc_edge_narrow(src, dst, sd3[:, 0], sd3[:, 1], hext3)

    out = pl.pallas_call(
        _final_body, out_shape=_f32((NP, N_CLASSES)),
    )(acc3, b3.reshape(1, -1))
    return out[:N_NODES]


# speedup vs baseline: 1.3463x; 1.3463x over previous
import functools

import jax
import jax.numpy as jnp
from jax import lax
from jax.experimental import pallas as pl
from jax.experimental.pallas import tpu as pltpu
from jax.experimental.pallas import tpu_sc as plsc

N_NODES = 10000
NP = 10240
E_RAW = 320000
E_TOT = E_RAW + N_NODES
NWORK = 32
EDGE_BLK = 128
NBLK_A = 52
NBLK_B = 110
TOT_BLOCKS = 16 * (NBLK_A + NBLK_B)
E_PAD = TOT_BLOCKS * EDGE_BLK
NEG = -1e30
D_FEAT = 128
NHID = 64
N_CLASSES = 16


WF = 128


def _make_sc_edge_kernel(nscale):
    mesh = plsc.VectorSubcoreMesh(core_axis_name="c", subcore_axis_name="s")
    rows_per_sub = NP // 16

    @functools.partial(
        pl.kernel,
        out_type=jax.ShapeDtypeStruct((2, NP, WF), jnp.float32),
        mesh=mesh,
        scratch_types=[
            pltpu.VMEM((EDGE_BLK,), jnp.int32),
            pltpu.VMEM((EDGE_BLK,), jnp.int32),
            pltpu.VMEM((EDGE_BLK,), jnp.int32),
            pltpu.VMEM((EDGE_BLK,), jnp.int32),
            pltpu.VMEM((EDGE_BLK,), jnp.float32),
            pltpu.VMEM((EDGE_BLK,), jnp.float32),
            pltpu.VMEM((EDGE_BLK,), jnp.float32),
            pltpu.VMEM((EDGE_BLK,), jnp.float32),
            pltpu.VMEM((EDGE_BLK,), jnp.float32),
            pltpu.VMEM((EDGE_BLK, WF), jnp.float32),
            pltpu.VMEM((EDGE_BLK, WF), jnp.float32),
            pltpu.VMEM_SHARED((NP,), jnp.float32),
            pltpu.VMEM_SHARED((NP,), jnp.float32),
            pltpu.VMEM_SHARED((NP, WF), jnp.float32),
            pltpu.SemaphoreType.DMA,
            pltpu.SemaphoreType.DMA,
            pltpu.SemaphoreType.DMA,
            pltpu.SemaphoreType.DMA,
            pltpu.SemaphoreType.DMA,
            pltpu.SemaphoreType.DMA,
            pltpu.SemaphoreType.DMA,
            pltpu.SemaphoreType.DMA,
        ],
        compiler_params=pltpu.CompilerParams(needs_layout_passes=False),
    )
    def k(src_hbm, dst_hbm, s_hbm, d_hbm, hext_hbm, acc_hbm,
          sidx0, sidx1, didx0, didx1, sv0, sv1, dv0, dv1, w_v,
          rows0, rows1, s_sh, d_sh, acc_sh,
          isem0, isem1, csem0, csem1, gsem0, gsem1, ssem0, ssem1):
        cid = lax.axis_index("c")
        sid = lax.axis_index("s")
        base = jnp.where(cid == 0, sid * NBLK_A,
                         16 * NBLK_A + sid * NBLK_B)
        nb = jnp.where(cid == 0, NBLK_A, NBLK_B)
        sidx = (sidx0, sidx1)
        didx = (didx0, didx1)
        sv = (sv0, sv1)
        dv = (dv0, dv1)
        rows = (rows0, rows1)
        isem = (isem0, isem1)
        csem = (csem0, csem1)
        gsem = (gsem0, gsem1)
        ssem = (ssem0, ssem1)

        @pl.when(sid == 0)
        def _():
            pltpu.sync_copy(s_hbm, s_sh)
            pltpu.sync_copy(d_hbm, d_sh)

        @pl.loop(0, EDGE_BLK)
        def _(r):
            for c in range(WF // 16):
                rows0[r, pl.ds(c * 16, 16)] = jnp.zeros((16,), jnp.float32)

        @pl.loop(0, rows_per_sub, step=EDGE_BLK)
        def _(r0):
            pltpu.sync_copy(
                rows0, acc_sh.at[pl.ds(sid * rows_per_sub + r0, EDGE_BLK)])

        plsc.subcore_barrier()

        def issue_idx(tt, b):
            pltpu.async_copy(src_hbm.at[base + tt], sidx[b], isem[b])
            pltpu.async_copy(dst_hbm.at[base + tt], didx[b], isem[b])

        def wait_idx(tt, b):
            pltpu.make_async_copy(
                src_hbm.at[base + tt], sidx[b], isem[b]).wait()
            pltpu.make_async_copy(
                dst_hbm.at[base + tt], didx[b], isem[b]).wait()

        def issue_scores(b):
            pltpu.async_copy(s_sh.at[sidx[b]], sv[b], csem[b])
            pltpu.async_copy(d_sh.at[didx[b]], dv[b], csem[b])

        def wait_scores(b):
            pltpu.make_async_copy(s_sh.at[sidx[b]], sv[b], csem[b]).wait()
            pltpu.make_async_copy(d_sh.at[didx[b]], dv[b], csem[b]).wait()

        def issue_gather(b):
            pltpu.async_copy(hext_hbm.at[sidx[b]], rows[b], gsem[b])

        def wait_gather(b):
            pltpu.make_async_copy(
                hext_hbm.at[sidx[b]], rows[b], gsem[b]).wait()

        def issue_scatter(b):
            pltpu.async_copy(rows[b], acc_sh.at[didx[b]], ssem[b], add=True)

        def wait_scatter(b):
            pltpu.make_async_copy(
                rows[b], acc_sh.at[didx[b]], ssem[b]).wait()

        def block_body(tt, b, first=False, last=False):
            ob = 1 - b
            if not first:
                wait_scatter(ob)
            if not last:
                issue_idx(tt + 1, ob)
            wait_scores(b)

            @pl.loop(0, EDGE_BLK, step=16)
            def _(j):
                e = sv[b][pl.ds(j, 16)] + dv[b][pl.ds(j, 16)]
                e = jnp.where(e >= 0.0, e, 0.2 * e)
                w_v[pl.ds(j, 16)] = jnp.exp(e)

            wait_gather(b)
            if not last:
                wait_idx(tt + 1, ob)
                issue_scores(ob)
                issue_gather(ob)

            @pl.loop(0, EDGE_BLK, step=4)
            def _(r):
                for u in range(4):
                    wv = plsc.load_gather(
                        w_v, [jnp.full((16,), r + u, jnp.int32)])
                    for c in range(nscale):
                        sl = pl.ds(c * 16, 16)
                        rows[b][r + u, sl] = rows[b][r + u, sl] * wv

            issue_scatter(b)

        issue_idx(0, 0)
        wait_idx(0, 0)
        issue_scores(0)
        issue_gather(0)

        block_body(0, 0, first=True)

        @pl.loop(1, nb - 1, step=2)
        def _(t):
            block_body(t, 1)
            block_body(t + 1, 0)

        block_body(nb - 1, 1, last=True)
        wait_scatter(1)

        plsc.subcore_barrier()

        pltpu.sync_copy(
            acc_sh.at[pl.ds(sid * rows_per_sub, rows_per_sub)],
            acc_hbm.at[cid].at[pl.ds(sid * rows_per_sub, rows_per_sub)])

    return k


_sc_edge_wide = _make_sc_edge_kernel(5)
_sc_edge_narrow = _make_sc_edge_kernel(2)


def _valid_mask():
    return lax.broadcasted_iota(jnp.int32, (NP, 1), 0) < N_NODES


def _emit_layer_outputs(h, f_out, hext_ref, sd_ref, a):
    valid = _valid_mask()
    sd = jnp.dot(h, a, preferred_element_type=jnp.float32)
    hext_ref[:, :f_out] = jnp.where(valid, h, 0.0)
    col = lax.broadcasted_iota(jnp.int32, (NP, WF - f_out), 1)
    hext_ref[:, f_out:] = jnp.where(valid & (col == 0), 1.0, 0.0)
    sd_ref[...] = jnp.where(valid, sd, NEG)


def _dense1_body(xp_ref, w_ref, a_ref, hext_ref, sd_ref):
    h = jnp.dot(xp_ref[...], w_ref[...], preferred_element_type=jnp.float32)
    _emit_layer_outputs(h, NHID, hext_ref, sd_ref, a_ref[...])


def _make_combine_body(f_in, f_out):
    def body(acc_ref, b_ref, w_ref, a_ref, hext_ref, sd_ref):
        g = acc_ref[0] + acc_ref[1]
        num = g[:, :f_in]
        den = g[:, f_in:f_in + 1] + 1e-16
        xn = jnp.maximum(num / den + b_ref[...], 0.0)
        xn = jnp.where(_valid_mask(), xn, 0.0)
        h = jnp.dot(xn, w_ref[...], preferred_element_type=jnp.float32)
        _emit_layer_outputs(h, f_out, hext_ref, sd_ref, a_ref[...])
    return body


def _final_body(acc_ref, b_ref, out_ref):
    g = acc_ref[0] + acc_ref[1]
    o = g[:, :N_CLASSES] / (g[:, N_CLASSES:N_CLASSES + 1] + 1e-16) + b_ref[...]
    m = jnp.max(o, axis=1, keepdims=True)
    z = o - m
    out_ref[...] = z - jnp.log(jnp.sum(jnp.exp(z), axis=1, keepdims=True))


def _f32(shape):
    return jax.ShapeDtypeStruct(shape, jnp.float32)


def kernel(x, edge_index, W1, a1s, a1d, b1, W2, a2s, a2d, b2,
           W3, a3s, a3d, b3):
    ei = edge_index.astype(jnp.int32)
    loops = jnp.arange(N_NODES, dtype=jnp.int32)
    pad_src = jnp.full((E_PAD - E_TOT,), N_NODES, jnp.int32)
    pad_dst = N_NODES + (jnp.arange(E_PAD - E_TOT, dtype=jnp.int32)
                         % (NP - N_NODES))
    src = jnp.concatenate([ei[0], loops, pad_src]).reshape(
        TOT_BLOCKS, EDGE_BLK)
    dst = jnp.concatenate([ei[1], loops, pad_dst]).reshape(
        TOT_BLOCKS, EDGE_BLK)

    xp = jnp.pad(x, ((0, NP - N_NODES), (0, 0)))
    A1 = jnp.stack([a1s, a1d], axis=1)
    A2 = jnp.stack([a2s, a2d], axis=1)
    A3 = jnp.stack([a3s, a3d], axis=1)

    hext1, sd1 = pl.pallas_call(
        _dense1_body, out_shape=(_f32((NP, WF)), _f32((NP, 2))),
    )(xp, W1, A1)
    acc1 = _sc_edge_wide(src, dst, sd1[:, 0], sd1[:, 1], hext1)

    hext2, sd2 = pl.pallas_call(
        _make_combine_body(NHID, NHID),
        out_shape=(_f32((NP, WF)), _f32((NP, 2))),
    )(acc1, b1.reshape(1, -1), W2, A2)
    acc2 = _sc_edge_wide(src, dst, sd2[:, 0], sd2[:, 1], hext2)

    hext3, sd3 = pl.pallas_call(
        _make_combine_body(NHID, N_CLASSES),
        out_shape=(_f32((NP, WF)), _f32((NP, 2))),
    )(acc2, b2.reshape(1, -1), W3, A3)
    acc3 = _sc_edge_narrow(src, dst, sd3[:, 0], sd3[:, 1], hext3)

    out = pl.pallas_call(
        _final_body, out_shape=_f32((NP, N_CLASSES)),
    )(acc3, b3.reshape(1, -1))
    return out[:N_NODES]


# speedup vs baseline: 1.7017x; 1.2639x over previous
import functools

import jax
import jax.numpy as jnp
from jax import lax
from jax.experimental import pallas as pl
from jax.experimental.pallas import tpu as pltpu
from jax.experimental.pallas import tpu_sc as plsc

N_NODES = 10000
NP = 10240
E_RAW = 320000
E_TOT = E_RAW + N_NODES
NWORK = 32
EDGE_BLK = 128
NBLK_A = 110
NBLK_B = 52
TOT_BLOCKS = 16 * (NBLK_A + NBLK_B)
E_PAD = TOT_BLOCKS * EDGE_BLK
NEG = -1e30
D_FEAT = 128
NHID = 64
N_CLASSES = 16


WF = 128


def _make_sc_edge_kernel(nscale):
    mesh = plsc.VectorSubcoreMesh(core_axis_name="c", subcore_axis_name="s")
    rows_per_sub = NP // 16

    @functools.partial(
        pl.kernel,
        out_type=jax.ShapeDtypeStruct((2, NP, WF), jnp.float32),
        mesh=mesh,
        scratch_types=[
            pltpu.VMEM((EDGE_BLK,), jnp.int32),
            pltpu.VMEM((EDGE_BLK,), jnp.int32),
            pltpu.VMEM((EDGE_BLK,), jnp.int32),
            pltpu.VMEM((EDGE_BLK,), jnp.int32),
            pltpu.VMEM((EDGE_BLK,), jnp.float32),
            pltpu.VMEM((EDGE_BLK,), jnp.float32),
            pltpu.VMEM((EDGE_BLK,), jnp.float32),
            pltpu.VMEM((EDGE_BLK,), jnp.float32),
            pltpu.VMEM((EDGE_BLK,), jnp.float32),
            pltpu.VMEM((EDGE_BLK, WF), jnp.float32),
            pltpu.VMEM((EDGE_BLK, WF), jnp.float32),
            pltpu.VMEM_SHARED((NP,), jnp.float32),
            pltpu.VMEM_SHARED((NP,), jnp.float32),
            pltpu.VMEM_SHARED((NP, WF), jnp.float32),
            pltpu.SemaphoreType.DMA,
            pltpu.SemaphoreType.DMA,
            pltpu.SemaphoreType.DMA,
            pltpu.SemaphoreType.DMA,
            pltpu.SemaphoreType.DMA,
            pltpu.SemaphoreType.DMA,
            pltpu.SemaphoreType.DMA,
            pltpu.SemaphoreType.DMA,
        ],
        compiler_params=pltpu.CompilerParams(needs_layout_passes=False),
    )
    def k(src_hbm, dst_hbm, s_hbm, d_hbm, hext_hbm, acc_hbm,
          sidx0, sidx1, didx0, didx1, sv0, sv1, dv0, dv1, w_v,
          rows0, rows1, s_sh, d_sh, acc_sh,
          isem0, isem1, csem0, csem1, gsem0, gsem1, ssem0, ssem1):
        cid = lax.axis_index("c")
        sid = lax.axis_index("s")
        base = jnp.where(cid == 0, sid * NBLK_A,
                         16 * NBLK_A + sid * NBLK_B)
        nb = jnp.where(cid == 0, NBLK_A, NBLK_B)
        sidx = (sidx0, sidx1)
        didx = (didx0, didx1)
        sv = (sv0, sv1)
        dv = (dv0, dv1)
        rows = (rows0, rows1)
        isem = (isem0, isem1)
        csem = (csem0, csem1)
        gsem = (gsem0, gsem1)
        ssem = (ssem0, ssem1)

        @pl.when(sid == 0)
        def _():
            pltpu.sync_copy(s_hbm, s_sh)
            pltpu.sync_copy(d_hbm, d_sh)

        @pl.loop(0, EDGE_BLK)
        def _(r):
            for c in range(WF // 16):
                rows0[r, pl.ds(c * 16, 16)] = jnp.zeros((16,), jnp.float32)

        @pl.loop(0, rows_per_sub, step=EDGE_BLK)
        def _(r0):
            pltpu.sync_copy(
                rows0, acc_sh.at[pl.ds(sid * rows_per_sub + r0, EDGE_BLK)])

        plsc.subcore_barrier()

        def issue_idx(tt, b):
            pltpu.async_copy(src_hbm.at[base + tt], sidx[b], isem[b])
            pltpu.async_copy(dst_hbm.at[base + tt], didx[b], isem[b])

        def wait_idx(tt, b):
            pltpu.make_async_copy(
                src_hbm.at[base + tt], sidx[b], isem[b]).wait()
            pltpu.make_async_copy(
                dst_hbm.at[base + tt], didx[b], isem[b]).wait()

        def issue_scores(b):
            pltpu.async_copy(s_sh.at[sidx[b]], sv[b], csem[b])
            pltpu.async_copy(d_sh.at[didx[b]], dv[b], csem[b])

        def wait_scores(b):
            pltpu.make_async_copy(s_sh.at[sidx[b]], sv[b], csem[b]).wait()
            pltpu.make_async_copy(d_sh.at[didx[b]], dv[b], csem[b]).wait()

        def issue_gather(b):
            pltpu.async_copy(hext_hbm.at[sidx[b]], rows[b], gsem[b])

        def wait_gather(b):
            pltpu.make_async_copy(
                hext_hbm.at[sidx[b]], rows[b], gsem[b]).wait()

        def issue_scatter(b):
            pltpu.async_copy(rows[b], acc_sh.at[didx[b]], ssem[b], add=True)

        def wait_scatter(b):
            pltpu.make_async_copy(
                rows[b], acc_sh.at[didx[b]], ssem[b]).wait()

        def block_body(tt, b, first=False, last=False):
            ob = 1 - b
            if not first:
                wait_scatter(ob)
            if not last:
                issue_idx(tt + 1, ob)
            wait_scores(b)

            @pl.loop(0, EDGE_BLK, step=16)
            def _(j):
                e = sv[b][pl.ds(j, 16)] + dv[b][pl.ds(j, 16)]
                e = jnp.where(e >= 0.0, e, 0.2 * e)
                w_v[pl.ds(j, 16)] = jnp.exp(e)

            wait_gather(b)
            if not last:
                wait_idx(tt + 1, ob)
                issue_scores(ob)
                issue_gather(ob)

            @pl.loop(0, EDGE_BLK, step=4)
            def _(r):
                for u in range(4):
                    wv = plsc.load_gather(
                        w_v, [jnp.full((16,), r + u, jnp.int32)])
                    for c in range(nscale):
                        sl = pl.ds(c * 16, 16)
                        rows[b][r + u, sl] = rows[b][r + u, sl] * wv

            issue_scatter(b)

        issue_idx(0, 0)
        wait_idx(0, 0)
        issue_scores(0)
        issue_gather(0)

        block_body(0, 0, first=True)

        @pl.loop(1, nb - 1, step=2)
        def _(t):
            block_body(t, 1)
            block_body(t + 1, 0)

        block_body(nb - 1, 1, last=True)
        wait_scatter(1)

        plsc.subcore_barrier()

        pltpu.sync_copy(
            acc_sh.at[pl.ds(sid * rows_per_sub, rows_per_sub)],
            acc_hbm.at[cid].at[pl.ds(sid * rows_per_sub, rows_per_sub)])

    return k


_sc_edge_wide = _make_sc_edge_kernel(5)
_sc_edge_narrow = _make_sc_edge_kernel(2)


def _valid_mask():
    return lax.broadcasted_iota(jnp.int32, (NP, 1), 0) < N_NODES


def _emit_layer_outputs(h, f_out, hext_ref, sd_ref, a):
    valid = _valid_mask()
    sd = jnp.dot(h, a, preferred_element_type=jnp.float32)
    hext_ref[:, :f_out] = jnp.where(valid, h, 0.0)
    col = lax.broadcasted_iota(jnp.int32, (NP, WF - f_out), 1)
    hext_ref[:, f_out:] = jnp.where(valid & (col == 0), 1.0, 0.0)
    sd_ref[...] = jnp.where(valid, sd, NEG)


def _dense1_body(xp_ref, w_ref, a_ref, hext_ref, sd_ref):
    h = jnp.dot(xp_ref[...], w_ref[...], preferred_element_type=jnp.float32)
    _emit_layer_outputs(h, NHID, hext_ref, sd_ref, a_ref[...])


def _make_combine_body(f_in, f_out):
    def body(acc_ref, b_ref, w_ref, a_ref, hext_ref, sd_ref):
        g = acc_ref[0] + acc_ref[1]
        num = g[:, :f_in]
        den = g[:, f_in:f_in + 1] + 1e-16
        xn = jnp.maximum(num / den + b_ref[...], 0.0)
        xn = jnp.where(_valid_mask(), xn, 0.0)
        h = jnp.dot(xn, w_ref[...], preferred_element_type=jnp.float32)
        _emit_layer_outputs(h, f_out, hext_ref, sd_ref, a_ref[...])
    return body


def _final_body(acc_ref, b_ref, out_ref):
    g = acc_ref[0] + acc_ref[1]
    o = g[:, :N_CLASSES] / (g[:, N_CLASSES:N_CLASSES + 1] + 1e-16) + b_ref[...]
    m = jnp.max(o, axis=1, keepdims=True)
    z = o - m
    out_ref[...] = z - jnp.log(jnp.sum(jnp.exp(z), axis=1, keepdims=True))


def _f32(shape):
    return jax.ShapeDtypeStruct(shape, jnp.float32)


def kernel(x, edge_index, W1, a1s, a1d, b1, W2, a2s, a2d, b2,
           W3, a3s, a3d, b3):
    ei = edge_index.astype(jnp.int32)
    loops = jnp.arange(N_NODES, dtype=jnp.int32)
    pad_src = jnp.full((E_PAD - E_TOT,), N_NODES, jnp.int32)
    pad_dst = N_NODES + (jnp.arange(E_PAD - E_TOT, dtype=jnp.int32)
                         % (NP - N_NODES))
    src = jnp.concatenate([ei[0], loops, pad_src]).reshape(
        TOT_BLOCKS, EDGE_BLK)
    dst = jnp.concatenate([ei[1], loops, pad_dst]).reshape(
        TOT_BLOCKS, EDGE_BLK)

    xp = jnp.pad(x, ((0, NP - N_NODES), (0, 0)))
    A1 = jnp.stack([a1s, a1d], axis=1)
    A2 = jnp.stack([a2s, a2d], axis=1)
    A3 = jnp.stack([a3s, a3d], axis=1)

    hext1, sd1 = pl.pallas_call(
        _dense1_body, out_shape=(_f32((NP, WF)), _f32((NP, 2))),
    )(xp, W1, A1)
    acc1 = _sc_edge_wide(src, dst, sd1[:, 0], sd1[:, 1], hext1)

    hext2, sd2 = pl.pallas_call(
        _make_combine_body(NHID, NHID),
        out_shape=(_f32((NP, WF)), _f32((NP, 2))),
    )(acc1, b1.reshape(1, -1), W2, A2)
    acc2 = _sc_edge_wide(src, dst, sd2[:, 0], sd2[:, 1], hext2)

    hext3, sd3 = pl.pallas_call(
        _make_combine_body(NHID, N_CLASSES),
        out_shape=(_f32((NP, WF)), _f32((NP, 2))),
    )(acc2, b2.reshape(1, -1), W3, A3)
    acc3 = _sc_edge_narrow(src, dst, sd3[:, 0], sd3[:, 1], hext3)

    out = pl.pallas_call(
        _final_body, out_shape=_f32((NP, N_CLASSES)),
    )(acc3, b3.reshape(1, -1))
    return out[:N_NODES]


# speedup vs baseline: 1.7996x; 1.0576x over previous
import functools

import jax
import jax.numpy as jnp
from jax import lax
from jax.experimental import pallas as pl
from jax.experimental.pallas import tpu as pltpu
from jax.experimental.pallas import tpu_sc as plsc

N_NODES = 10000
NP = 10240
E_RAW = 320000
E_TOT = E_RAW + N_NODES
NWORK = 32
EDGE_BLK = 128
NBLK_A = 102
NBLK_B = 60
TOT_BLOCKS = 16 * (NBLK_A + NBLK_B)
E_PAD = TOT_BLOCKS * EDGE_BLK
NEG = -1e30
D_FEAT = 128
NHID = 64
N_CLASSES = 16


WF = 128


def _make_sc_edge_kernel(nscale):
    mesh = plsc.VectorSubcoreMesh(core_axis_name="c", subcore_axis_name="s")
    rows_per_sub = NP // 16

    @functools.partial(
        pl.kernel,
        out_type=jax.ShapeDtypeStruct((2, NP, WF), jnp.float32),
        mesh=mesh,
        scratch_types=[
            pltpu.VMEM((EDGE_BLK,), jnp.int32),
            pltpu.VMEM((EDGE_BLK,), jnp.int32),
            pltpu.VMEM((EDGE_BLK,), jnp.int32),
            pltpu.VMEM((EDGE_BLK,), jnp.int32),
            pltpu.VMEM((EDGE_BLK,), jnp.float32),
            pltpu.VMEM((EDGE_BLK,), jnp.float32),
            pltpu.VMEM((EDGE_BLK,), jnp.float32),
            pltpu.VMEM((EDGE_BLK,), jnp.float32),
            pltpu.VMEM((EDGE_BLK,), jnp.float32),
            pltpu.VMEM((EDGE_BLK, WF), jnp.float32),
            pltpu.VMEM((EDGE_BLK, WF), jnp.float32),
            pltpu.VMEM_SHARED((NP,), jnp.float32),
            pltpu.VMEM_SHARED((NP,), jnp.float32),
            pltpu.VMEM_SHARED((NP, WF), jnp.float32),
            pltpu.SemaphoreType.DMA,
            pltpu.SemaphoreType.DMA,
            pltpu.SemaphoreType.DMA,
            pltpu.SemaphoreType.DMA,
            pltpu.SemaphoreType.DMA,
            pltpu.SemaphoreType.DMA,
            pltpu.SemaphoreType.DMA,
            pltpu.SemaphoreType.DMA,
        ],
        compiler_params=pltpu.CompilerParams(needs_layout_passes=False),
    )
    def k(src_hbm, dst_hbm, s_hbm, d_hbm, hext_hbm, acc_hbm,
          sidx0, sidx1, didx0, didx1, sv0, sv1, dv0, dv1, w_v,
          rows0, rows1, s_sh, d_sh, acc_sh,
          isem0, isem1, csem0, csem1, gsem0, gsem1, ssem0, ssem1):
        cid = lax.axis_index("c")
        sid = lax.axis_index("s")
        base = jnp.where(cid == 0, sid * NBLK_A,
                         16 * NBLK_A + sid * NBLK_B)
        nb = jnp.where(cid == 0, NBLK_A, NBLK_B)
        sidx = (sidx0, sidx1)
        didx = (didx0, didx1)
        sv = (sv0, sv1)
        dv = (dv0, dv1)
        rows = (rows0, rows1)
        isem = (isem0, isem1)
        csem = (csem0, csem1)
        gsem = (gsem0, gsem1)
        ssem = (ssem0, ssem1)

        @pl.when(sid == 0)
        def _():
            pltpu.sync_copy(s_hbm, s_sh)
            pltpu.sync_copy(d_hbm, d_sh)

        @pl.loop(0, EDGE_BLK)
        def _(r):
            for c in range(WF // 16):
                rows0[r, pl.ds(c * 16, 16)] = jnp.zeros((16,), jnp.float32)

        @pl.loop(0, rows_per_sub, step=EDGE_BLK)
        def _(r0):
            pltpu.sync_copy(
                rows0, acc_sh.at[pl.ds(sid * rows_per_sub + r0, EDGE_BLK)])

        plsc.subcore_barrier()

        def issue_idx(tt, b):
            pltpu.async_copy(src_hbm.at[base + tt], sidx[b], isem[b])
            pltpu.async_copy(dst_hbm.at[base + tt], didx[b], isem[b])

        def wait_idx(tt, b):
            pltpu.make_async_copy(
                src_hbm.at[base + tt], sidx[b], isem[b]).wait()
            pltpu.make_async_copy(
                dst_hbm.at[base + tt], didx[b], isem[b]).wait()

        def issue_scores(b):
            pltpu.async_copy(s_sh.at[sidx[b]], sv[b], csem[b])
            pltpu.async_copy(d_sh.at[didx[b]], dv[b], csem[b])

        def wait_scores(b):
            pltpu.make_async_copy(s_sh.at[sidx[b]], sv[b], csem[b]).wait()
            pltpu.make_async_copy(d_sh.at[didx[b]], dv[b], csem[b]).wait()

        def issue_gather(b):
            pltpu.async_copy(hext_hbm.at[sidx[b]], rows[b], gsem[b])

        def wait_gather(b):
            pltpu.make_async_copy(
                hext_hbm.at[sidx[b]], rows[b], gsem[b]).wait()

        def issue_scatter(b):
            pltpu.async_copy(rows[b], acc_sh.at[didx[b]], ssem[b], add=True)

        def wait_scatter(b):
            pltpu.make_async_copy(
                rows[b], acc_sh.at[didx[b]], ssem[b]).wait()

        def block_body(tt, b, first=False, last=False):
            ob = 1 - b
            if not first:
                wait_scatter(ob)
            if not last:
                issue_idx(tt + 1, ob)
            wait_scores(b)

            @pl.loop(0, EDGE_BLK, step=16)
            def _(j):
                e = sv[b][pl.ds(j, 16)] + dv[b][pl.ds(j, 16)]
                e = jnp.where(e >= 0.0, e, 0.2 * e)
                w_v[pl.ds(j, 16)] = jnp.exp(e)

            wait_gather(b)
            if not last:
                wait_idx(tt + 1, ob)
                issue_scores(ob)
                issue_gather(ob)

            @pl.loop(0, EDGE_BLK, step=4)
            def _(r):
                for u in range(4):
                    wv = plsc.load_gather(
                        w_v, [jnp.full((16,), r + u, jnp.int32)])
                    for c in range(nscale):
                        sl = pl.ds(c * 16, 16)
                        rows[b][r + u, sl] = rows[b][r + u, sl] * wv

            issue_scatter(b)

        issue_idx(0, 0)
        wait_idx(0, 0)
        issue_scores(0)
        issue_gather(0)

        block_body(0, 0, first=True)

        @pl.loop(1, nb - 1, step=2)
        def _(t):
            block_body(t, 1)
            block_body(t + 1, 0)

        block_body(nb - 1, 1, last=True)
        wait_scatter(1)

        plsc.subcore_barrier()

        pltpu.sync_copy(
            acc_sh.at[pl.ds(sid * rows_per_sub, rows_per_sub)],
            acc_hbm.at[cid].at[pl.ds(sid * rows_per_sub, rows_per_sub)])

    return k


_sc_edge_wide = _make_sc_edge_kernel(5)
_sc_edge_narrow = _make_sc_edge_kernel(2)


def _valid_mask():
    return lax.broadcasted_iota(jnp.int32, (NP, 1), 0) < N_NODES


def _emit_layer_outputs(h, f_out, hext_ref, sd_ref, a):
    valid = _valid_mask()
    sd = jnp.dot(h, a, preferred_element_type=jnp.float32)
    hext_ref[:, :f_out] = jnp.where(valid, h, 0.0)
    col = lax.broadcasted_iota(jnp.int32, (NP, WF - f_out), 1)
    hext_ref[:, f_out:] = jnp.where(valid & (col == 0), 1.0, 0.0)
    sd_ref[...] = jnp.where(valid, sd, NEG)


def _dense1_body(xp_ref, w_ref, a_ref, hext_ref, sd_ref):
    h = jnp.dot(xp_ref[...], w_ref[...], preferred_element_type=jnp.float32)
    _emit_layer_outputs(h, NHID, hext_ref, sd_ref, a_ref[...])


def _make_combine_body(f_in, f_out):
    def body(acc_ref, b_ref, w_ref, a_ref, hext_ref, sd_ref):
        g = acc_ref[0] + acc_ref[1]
        num = g[:, :f_in]
        den = g[:, f_in:f_in + 1] + 1e-16
        xn = jnp.maximum(num / den + b_ref[...], 0.0)
        xn = jnp.where(_valid_mask(), xn, 0.0)
        h = jnp.dot(xn, w_ref[...], preferred_element_type=jnp.float32)
        _emit_layer_outputs(h, f_out, hext_ref, sd_ref, a_ref[...])
    return body


def _final_body(acc_ref, b_ref, out_ref):
    g = acc_ref[0] + acc_ref[1]
    o = g[:, :N_CLASSES] / (g[:, N_CLASSES:N_CLASSES + 1] + 1e-16) + b_ref[...]
    m = jnp.max(o, axis=1, keepdims=True)
    z = o - m
    out_ref[...] = z - jnp.log(jnp.sum(jnp.exp(z), axis=1, keepdims=True))


def _f32(shape):
    return jax.ShapeDtypeStruct(shape, jnp.float32)


def kernel(x, edge_index, W1, a1s, a1d, b1, W2, a2s, a2d, b2,
           W3, a3s, a3d, b3):
    ei = edge_index.astype(jnp.int32)
    loops = jnp.arange(N_NODES, dtype=jnp.int32)
    pad_src = jnp.full((E_PAD - E_TOT,), N_NODES, jnp.int32)
    pad_dst = N_NODES + (jnp.arange(E_PAD - E_TOT, dtype=jnp.int32)
                         % (NP - N_NODES))
    src = jnp.concatenate([ei[0], loops, pad_src]).reshape(
        TOT_BLOCKS, EDGE_BLK)
    dst = jnp.concatenate([ei[1], loops, pad_dst]).reshape(
        TOT_BLOCKS, EDGE_BLK)

    xp = jnp.pad(x, ((0, NP - N_NODES), (0, 0)))
    A1 = jnp.stack([a1s, a1d], axis=1)
    A2 = jnp.stack([a2s, a2d], axis=1)
    A3 = jnp.stack([a3s, a3d], axis=1)

    hext1, sd1 = pl.pallas_call(
        _dense1_body, out_shape=(_f32((NP, WF)), _f32((NP, 2))),
    )(xp, W1, A1)
    acc1 = _sc_edge_wide(src, dst, sd1[:, 0], sd1[:, 1], hext1)

    hext2, sd2 = pl.pallas_call(
        _make_combine_body(NHID, NHID),
        out_shape=(_f32((NP, WF)), _f32((NP, 2))),
    )(acc1, b1.reshape(1, -1), W2, A2)
    acc2 = _sc_edge_wide(src, dst, sd2[:, 0], sd2[:, 1], hext2)

    hext3, sd3 = pl.pallas_call(
        _make_combine_body(NHID, N_CLASSES),
        out_shape=(_f32((NP, WF)), _f32((NP, 2))),
    )(acc2, b2.reshape(1, -1), W3, A3)
    acc3 = _sc_edge_narrow(src, dst, sd3[:, 0], sd3[:, 1], hext3)

    out = pl.pallas_call(
        _final_body, out_shape=_f32((NP, N_CLASSES)),
    )(acc3, b3.reshape(1, -1))
    return out[:N_NODES]
